# baseline (speedup 1.0000x reference)
import jax
import jax.numpy as jnp
from jax import lax
from jax.experimental import pallas as pl
from jax.experimental.pallas import tpu as pltpu
from jax.experimental.pallas import tpu_sc as plsc

NC = 2
NS = 16
NW = NC * NS
LANES = 128
SLOT = 64
PADROWS = 56
NBUF = 4
VL = 16


def _mesh():
    return plsc.VectorSubcoreMesh(
        core_axis_name="c", subcore_axis_name="s", num_cores=NC, num_subcores=NS
    )


def _wid():
    return lax.axis_index("s") * NC + lax.axis_index("c")


def _repack_body(n_l, xp_hbm, idx3_hbm, slab_v, flat_v, sem):
    n_b = xp_hbm.shape[0] // NW
    n_rows = idx3_hbm.shape[1]
    wid = _wid()

    pltpu.sync_copy(xp_hbm.at[pl.ds(wid * n_b, n_b)], slab_v)

    ks = list(range(0, n_l - VL, VL)) + [n_l - VL]
    zeros = jnp.zeros((VL,), jnp.int32)

    def rloop(r, carry):
        flat_v[pl.ds(SLOT * r + SLOT - VL, VL)] = zeros
        for k in ks:
            flat_v[pl.ds(SLOT * r + k, VL)] = slab_v[r, pl.ds(k, VL)]
        return carry

    lax.fori_loop(0, n_b, rloop, 0)

    def wstart(rr, carry):
        pltpu.async_copy(
            flat_v.at[pl.ds(LANES * rr, LANES)], idx3_hbm.at[wid, rr], sem
        )
        return carry

    lax.fori_loop(0, n_rows, wstart, 0)

    def wdrain(rr, carry):
        pltpu.make_async_copy(
            flat_v.at[pl.ds(LANES * rr, LANES)], idx3_hbm.at[wid, rr], sem
        ).wait()
        return carry

    lax.fori_loop(0, n_rows, wdrain, 0)


def _gather_body(table_hbm, idx3_hbm, out_hbm, idx_v, rows_v, gsems):
    n_b = out_hbm.shape[0] // NW
    wid = _wid()
    b0 = wid * n_b

    pltpu.sync_copy(idx3_hbm.at[wid], idx_v)

    def idx_slice(r):
        off = pl.multiple_of(lax.rem(r, 2) * SLOT, SLOT)
        return idx_v.at[r // 2, pl.ds(off, SLOT)]

    def start_gather(r, buf):
        pltpu.async_copy(
            table_hbm.at[idx_slice(r)], rows_v.at[buf], gsems.at[buf]
        )

    for r in range(NBUF - 1):
        start_gather(r, r)

    def step(r, carry):
        buf = lax.rem(r, NBUF)

        @pl.when(r + NBUF - 1 < n_b)
        def _():
            start_gather(r + NBUF - 1, lax.rem(r + NBUF - 1, NBUF))

        pltpu.make_async_copy(
            table_hbm.at[idx_slice(r)], rows_v.at[buf], gsems.at[buf]
        ).wait()
        pltpu.sync_copy(
            rows_v.at[buf, pl.ds(0, PADROWS)], out_hbm.at[b0 + r]
        )
        return carry

    lax.fori_loop(0, n_b, step, 0)


def kernel(x, table):
    b, l = x.shape
    emb = table.shape[1]
    assert b % NW == 0
    n_b = b // NW

    xp = jnp.pad(x.astype(jnp.int32), ((0, 0), (0, LANES - l)))
    tp = jnp.pad(table, ((0, 0), (0, LANES - emb)))

    repack = pl.kernel(
        lambda *args: _repack_body(l, *args),
        out_type=jax.ShapeDtypeStruct((NW, n_b * SLOT // LANES, LANES), jnp.int32),
        mesh=_mesh(),
        scratch_types=[
            pltpu.VMEM((n_b, LANES), jnp.int32),
            pltpu.VMEM((n_b * SLOT,), jnp.int32),
            pltpu.SemaphoreType.DMA,
        ],
        compiler_params=pltpu.CompilerParams(use_tc_tiling_on_sc=True),
    )
    idx3 = repack(xp)

    gather = pl.kernel(
        _gather_body,
        out_type=jax.ShapeDtypeStruct((b, PADROWS, LANES), jnp.float32),
        mesh=_mesh(),
        scratch_types=[
            pltpu.VMEM((n_b * SLOT // LANES, LANES), jnp.int32),
            pltpu.VMEM((NBUF, SLOT, LANES), jnp.float32),
            pltpu.SemaphoreType.DMA((NBUF,)),
        ],
        compiler_params=pltpu.CompilerParams(use_tc_tiling_on_sc=False),
    )
    out = gather(tp, idx3)
    return out[:, :l, :emb]


# speedup vs baseline: 1.0006x; 1.0006x over previous
import jax
import jax.numpy as jnp
from jax import lax
from jax.experimental import pallas as pl
from jax.experimental.pallas import tpu as pltpu
from jax.experimental.pallas import tpu_sc as plsc

NC = 2
NS = 16
NW = NC * NS
LANES = 128
SLOT = 64
PADROWS = 56
NBUF = 4
VL = 16


def _mesh():
    return plsc.VectorSubcoreMesh(
        core_axis_name="c", subcore_axis_name="s", num_cores=NC, num_subcores=NS
    )


def _wid():
    return lax.axis_index("s") * NC + lax.axis_index("c")


def _repack_body(n_l, xp_hbm, idx3_hbm, slab_v, flat_v, sem):
    n_b = xp_hbm.shape[0] // NW
    n_rows = idx3_hbm.shape[1]
    wid = _wid()

    pltpu.sync_copy(xp_hbm.at[pl.ds(wid * n_b, n_b)], slab_v)

    ks = list(range(0, n_l - VL, VL)) + [n_l - VL]
    zeros = jnp.zeros((VL,), jnp.int32)

    def rloop(r, carry):
        flat_v[pl.ds(SLOT * r + SLOT - VL, VL)] = zeros
        for k in ks:
            flat_v[pl.ds(SLOT * r + k, VL)] = slab_v[r, pl.ds(k, VL)]
        return carry

    lax.fori_loop(0, n_b, rloop, 0)

    def wstart(rr, carry):
        pltpu.async_copy(
            flat_v.at[pl.ds(LANES * rr, LANES)], idx3_hbm.at[wid, rr], sem
        )
        return carry

    lax.fori_loop(0, n_rows, wstart, 0)

    def wdrain(rr, carry):
        pltpu.make_async_copy(
            flat_v.at[pl.ds(LANES * rr, LANES)], idx3_hbm.at[wid, rr], sem
        ).wait()
        return carry

    lax.fori_loop(0, n_rows, wdrain, 0)


def _gather_body(table_hbm, idx3_hbm, out_hbm, idx_v, rows_v, gsems):
    n_rows = idx3_hbm.shape[1]
    wid = _wid()
    b0 = wid * (2 * n_rows)

    pltpu.sync_copy(idx3_hbm.at[wid], idx_v)

    def start_gather(j, buf):
        pltpu.async_copy(
            table_hbm.at[idx_v.at[j]], rows_v.at[buf], gsems.at[buf]
        )

    for j in range(NBUF - 1):
        start_gather(j, j)

    def step(j, carry):
        buf = lax.rem(j, NBUF)

        @pl.when(j + NBUF - 1 < n_rows)
        def _():
            start_gather(j + NBUF - 1, lax.rem(j + NBUF - 1, NBUF))

        pltpu.make_async_copy(
            table_hbm.at[idx_v.at[j]], rows_v.at[buf], gsems.at[buf]
        ).wait()
        pltpu.sync_copy(
            rows_v.at[buf, pl.ds(0, PADROWS)], out_hbm.at[b0 + 2 * j]
        )
        pltpu.sync_copy(
            rows_v.at[buf, pl.ds(SLOT, PADROWS)], out_hbm.at[b0 + 2 * j + 1]
        )
        return carry

    lax.fori_loop(0, n_rows, step, 0)


def kernel(x, table):
    b, l = x.shape
    emb = table.shape[1]
    assert b % NW == 0
    n_b = b // NW

    xp = jnp.pad(x.astype(jnp.int32), ((0, 0), (0, LANES - l)))
    tp = jnp.pad(table, ((0, 0), (0, LANES - emb)))

    repack = pl.kernel(
        lambda *args: _repack_body(l, *args),
        out_type=jax.ShapeDtypeStruct((NW, n_b * SLOT // LANES, LANES), jnp.int32),
        mesh=_mesh(),
        scratch_types=[
            pltpu.VMEM((n_b, LANES), jnp.int32),
            pltpu.VMEM((n_b * SLOT,), jnp.int32),
            pltpu.SemaphoreType.DMA,
        ],
        compiler_params=pltpu.CompilerParams(use_tc_tiling_on_sc=True),
    )
    idx3 = repack(xp)

    gather = pl.kernel(
        _gather_body,
        out_type=jax.ShapeDtypeStruct((b, PADROWS, LANES), jnp.float32),
        mesh=_mesh(),
        scratch_types=[
            pltpu.VMEM((n_b * SLOT // LANES, LANES), jnp.int32),
            pltpu.VMEM((NBUF, LANES, LANES), jnp.float32),
            pltpu.SemaphoreType.DMA((NBUF,)),
        ],
        compiler_params=pltpu.CompilerParams(use_tc_tiling_on_sc=False),
    )
    out = gather(tp, idx3)
    return out[:, :l, :emb]


# speedup vs baseline: 9.6591x; 9.6530x over previous
import jax
import jax.numpy as jnp
from jax import lax
from jax.experimental import pallas as pl
from jax.experimental.pallas import tpu as pltpu
from jax.experimental.pallas import tpu_sc as plsc

NC = 2
NS = 16
NW = NC * NS
LANES = 128
SLOT = 64
PADROWS = 56
NBUF = 4
VL = 16


def _mesh():
    return plsc.VectorSubcoreMesh(
        core_axis_name="c", subcore_axis_name="s", num_cores=NC, num_subcores=NS
    )


def _wid():
    return lax.axis_index("s") * NC + lax.axis_index("c")


def _repack_body(n_l, xp_hbm, idx3_hbm, slab_v, flat_v, sem):
    n_b = xp_hbm.shape[0] // NW
    n_rows = idx3_hbm.shape[1]
    wid = _wid()

    pltpu.sync_copy(xp_hbm.at[pl.ds(wid * n_b, n_b)], slab_v)

    ks = list(range(0, n_l - VL, VL)) + [n_l - VL]

    def rloop(r, carry):
        flat_v[pl.ds(SLOT * r + SLOT - VL, VL)] = slab_v[r, pl.ds(0, VL)]
        for k in ks:
            flat_v[pl.ds(SLOT * r + k, VL)] = slab_v[r, pl.ds(k, VL)]
        return carry

    lax.fori_loop(0, n_b, rloop, 0)

    def wstart(rr, carry):
        pltpu.async_copy(
            flat_v.at[pl.ds(LANES * rr, LANES)], idx3_hbm.at[wid, rr], sem
        )
        return carry

    lax.fori_loop(0, n_rows, wstart, 0)

    def wdrain(rr, carry):
        pltpu.make_async_copy(
            flat_v.at[pl.ds(LANES * rr, LANES)], idx3_hbm.at[wid, rr], sem
        ).wait()
        return carry

    lax.fori_loop(0, n_rows, wdrain, 0)


def _gather_body(table_hbm, idx3_hbm, out_hbm, idx_v, rows_v, gsems):
    n_rows = idx3_hbm.shape[1]
    wid = _wid()
    b0 = wid * (2 * n_rows)

    pltpu.sync_copy(idx3_hbm.at[wid], idx_v)

    def start_gather(j, buf):
        pltpu.async_copy(
            table_hbm.at[idx_v.at[j]], rows_v.at[buf], gsems.at[buf]
        )

    for j in range(NBUF - 1):
        start_gather(j, j)

    def step(j, carry):
        buf = lax.rem(j, NBUF)

        @pl.when(j + NBUF - 1 < n_rows)
        def _():
            start_gather(j + NBUF - 1, lax.rem(j + NBUF - 1, NBUF))

        pltpu.make_async_copy(
            table_hbm.at[idx_v.at[j]], rows_v.at[buf], gsems.at[buf]
        ).wait()
        pltpu.sync_copy(
            rows_v.at[buf, pl.ds(0, PADROWS)], out_hbm.at[b0 + 2 * j]
        )
        pltpu.sync_copy(
            rows_v.at[buf, pl.ds(SLOT, PADROWS)], out_hbm.at[b0 + 2 * j + 1]
        )
        return carry

    lax.fori_loop(0, n_rows, step, 0)


def kernel(x, table):
    b, l = x.shape
    emb = table.shape[1]
    assert b % NW == 0
    n_b = b // NW

    xp = jnp.pad(x.astype(jnp.int32), ((0, 0), (0, LANES - l)))
    tp = jnp.pad(table, ((0, 0), (0, LANES - emb)))

    repack = pl.kernel(
        lambda *args: _repack_body(l, *args),
        out_type=jax.ShapeDtypeStruct((NW, n_b * SLOT // LANES, LANES), jnp.int32),
        mesh=_mesh(),
        scratch_types=[
            pltpu.VMEM((n_b, LANES), jnp.int32),
            pltpu.VMEM((n_b * SLOT,), jnp.int32),
            pltpu.SemaphoreType.DMA,
        ],
        compiler_params=pltpu.CompilerParams(use_tc_tiling_on_sc=True),
    )
    idx3 = repack(xp)

    gather = pl.kernel(
        _gather_body,
        out_type=jax.ShapeDtypeStruct((b, PADROWS, LANES), jnp.float32),
        mesh=_mesh(),
        scratch_types=[
            pltpu.VMEM((n_b * SLOT // LANES, LANES), jnp.int32),
            pltpu.VMEM((NBUF, LANES, LANES), jnp.float32),
            pltpu.SemaphoreType.DMA((NBUF,)),
        ],
        compiler_params=pltpu.CompilerParams(use_tc_tiling_on_sc=False),
    )
    out = gather(tp, idx3)
    return out[:, :l, :emb]


# speedup vs baseline: 9.6639x; 1.0005x over previous
import jax
import jax.numpy as jnp
from jax import lax
from jax.experimental import pallas as pl
from jax.experimental.pallas import tpu as pltpu
from jax.experimental.pallas import tpu_sc as plsc

NC = 2
NS = 16
NW = NC * NS
LANES = 128
SLOT = 64
PADROWS = 56
NBUF = 5
VL = 16


def _mesh():
    return plsc.VectorSubcoreMesh(
        core_axis_name="c", subcore_axis_name="s", num_cores=NC, num_subcores=NS
    )


def _wid():
    return lax.axis_index("s") * NC + lax.axis_index("c")


def _repack_body(n_l, xp_hbm, idx3_hbm, slab_v, flat_v, sem):
    n_b = xp_hbm.shape[0] // NW
    n_rows = idx3_hbm.shape[1]
    wid = _wid()

    pltpu.sync_copy(xp_hbm.at[pl.ds(wid * n_b, n_b)], slab_v)

    ks = list(range(0, n_l - VL, VL)) + [n_l - VL]

    def rloop(r, carry):
        flat_v[pl.ds(SLOT * r + SLOT - VL, VL)] = slab_v[r, pl.ds(0, VL)]
        for k in ks:
            flat_v[pl.ds(SLOT * r + k, VL)] = slab_v[r, pl.ds(k, VL)]
        return carry

    lax.fori_loop(0, n_b, rloop, 0)

    def wstart(rr, carry):
        pltpu.async_copy(
            flat_v.at[pl.ds(LANES * rr, LANES)], idx3_hbm.at[wid, rr], sem
        )
        return carry

    lax.fori_loop(0, n_rows, wstart, 0)

    def wdrain(rr, carry):
        pltpu.make_async_copy(
            flat_v.at[pl.ds(LANES * rr, LANES)], idx3_hbm.at[wid, rr], sem
        ).wait()
        return carry

    lax.fori_loop(0, n_rows, wdrain, 0)


def _gather_body(table_hbm, idx3_hbm, out_hbm, idx_v, rows_v, gsems):
    n_rows = idx3_hbm.shape[1]
    wid = _wid()
    b0 = wid * (2 * n_rows)

    pltpu.sync_copy(idx3_hbm.at[wid], idx_v)

    def start_gather(j, buf):
        pltpu.async_copy(
            table_hbm.at[idx_v.at[j]], rows_v.at[buf], gsems.at[buf]
        )

    for j in range(NBUF - 1):
        start_gather(j, j)

    def step(j, carry):
        buf = lax.rem(j, NBUF)

        @pl.when(j + NBUF - 1 < n_rows)
        def _():
            start_gather(j + NBUF - 1, lax.rem(j + NBUF - 1, NBUF))

        pltpu.make_async_copy(
            table_hbm.at[idx_v.at[j]], rows_v.at[buf], gsems.at[buf]
        ).wait()
        pltpu.sync_copy(
            rows_v.at[buf, pl.ds(0, PADROWS)], out_hbm.at[b0 + 2 * j]
        )
        pltpu.sync_copy(
            rows_v.at[buf, pl.ds(SLOT, PADROWS)], out_hbm.at[b0 + 2 * j + 1]
        )
        return carry

    lax.fori_loop(0, n_rows, step, 0)


def kernel(x, table):
    b, l = x.shape
    emb = table.shape[1]
    assert b % NW == 0
    n_b = b // NW

    xp = jnp.pad(x.astype(jnp.int32), ((0, 0), (0, LANES - l)))
    tp = jnp.pad(table, ((0, 0), (0, LANES - emb)))

    repack = pl.kernel(
        lambda *args: _repack_body(l, *args),
        out_type=jax.ShapeDtypeStruct((NW, n_b * SLOT // LANES, LANES), jnp.int32),
        mesh=_mesh(),
        scratch_types=[
            pltpu.VMEM((n_b, LANES), jnp.int32),
            pltpu.VMEM((n_b * SLOT,), jnp.int32),
            pltpu.SemaphoreType.DMA,
        ],
        compiler_params=pltpu.CompilerParams(use_tc_tiling_on_sc=True),
    )
    idx3 = repack(xp)

    gather = pl.kernel(
        _gather_body,
        out_type=jax.ShapeDtypeStruct((b, PADROWS, LANES), jnp.float32),
        mesh=_mesh(),
        scratch_types=[
            pltpu.VMEM((n_b * SLOT // LANES, LANES), jnp.int32),
            pltpu.VMEM((NBUF, LANES, LANES), jnp.float32),
            pltpu.SemaphoreType.DMA((NBUF,)),
        ],
        compiler_params=pltpu.CompilerParams(use_tc_tiling_on_sc=False),
    )
    out = gather(tp, idx3)
    return out[:, :l, :emb]
